# Initial kernel scaffold; baseline (speedup 1.0000x reference)
import jax
import jax.numpy as jnp
from jax.experimental import pallas as pl


def kernel(resources, operations, need_edge_index, need_edge_attr, same_edge_index, W_self, W_res, W_op, att_op, att_res, att_self):
    raise NotImplementedError("write your pallas kernel here")


# speedup vs baseline: 2.1115x; 2.1115x over previous
import functools

import jax
import jax.numpy as jnp
from jax import lax
from jax.experimental import pallas as pl
from jax.experimental.pallas import tpu as pltpu
from jax.experimental.pallas import tpu_sc as plsc

N = 10000
E1 = 320000
E2 = 160000
D = 128
DE = 16

NC, NS, L = 2, 16, 16
NW = NC * NS

E1_W = E1 // NW
E2_W = E2 // NW
CH = 128
E1_WP = -(-E1_W // CH) * CH
E2_WP = -(-E2_W // CH) * CH
NCH1 = E1_WP // CH
NCH2 = E2_WP // CH
S1 = NW * E1_WP
S2 = NW * E2_WP
NP = 10240
ROWS_T = NP // NS

TOT = N + S1 + S2
SM_ROWS = 3888
TOTP = SM_ROWS * 128
SM_BLK = 1296

BLK = 400
EBLK = 4096


def _dense_body(r_ref, o_ref, wst_ref, wrt_ref, wot_ref,
                aS_ref, aO1_ref, aO2_ref, aR1_ref, aR2_ref,
                rp_ref, op_ref, scal_ref):
    r = r_ref[...]
    o = o_ref[...]
    sr = jnp.dot(r, wst_ref[...], preferred_element_type=jnp.float32)
    rp = jnp.dot(r, wrt_ref[...], preferred_element_type=jnp.float32)
    op = jnp.dot(o, wot_ref[...], preferred_element_type=jnp.float32)
    rp_ref[...] = rp
    op_ref[...] = op
    s_self = jnp.dot(sr, aS_ref[...])
    l_self = jnp.where(s_self > 0, s_self, 0.2 * s_self)
    a_res = jnp.dot(sr, aO1_ref[...])
    p = jnp.dot(sr, aR1_ref[...])
    q = jnp.dot(rp, aR2_ref[...])
    c_op = jnp.dot(op, aO2_ref[...])
    z = jnp.zeros_like(s_self)
    scal_ref[...] = jnp.concatenate([l_self, a_res, p, q, c_op, z, z, z], axis=1)


def _dense_call(resources, operations, wst, wrt, wot, aS, aO1, aO2, aR1, aR2):
    nblk = N // BLK
    full = lambda s: pl.BlockSpec(s, lambda i: (0, 0))
    return pl.pallas_call(
        _dense_body,
        grid=(nblk,),
        in_specs=[
            pl.BlockSpec((BLK, D), lambda i: (i, 0)),
            pl.BlockSpec((BLK, D), lambda i: (i, 0)),
            full((D, D)), full((D, D)), full((D, D)),
            full((D, 1)), full((D, 1)), full((D, 1)), full((D, 1)), full((D, 1)),
        ],
        out_specs=[
            pl.BlockSpec((BLK, D), lambda i: (i, 0)),
            pl.BlockSpec((BLK, D), lambda i: (i, 0)),
            pl.BlockSpec((BLK, 8), lambda i: (i, 0)),
        ],
        out_shape=[
            jax.ShapeDtypeStruct((N, D), jnp.float32),
            jax.ShapeDtypeStruct((N, D), jnp.float32),
            jax.ShapeDtypeStruct((N, 8), jnp.float32),
        ],
    )(resources, operations, wst, wrt, wot, aS, aO1, aO2, aR1, aR2)


def _attr_body(attr_ref, wet_ref, aO2_ref, ap_ref, d_ref):
    ap = jnp.dot(attr_ref[...], wet_ref[...], preferred_element_type=jnp.float32)
    ap_ref[...] = ap
    d_ref[...] = jnp.dot(ap, aO2_ref[...])


def _attr_call(attrp, wet, aO2):
    return pl.pallas_call(
        _attr_body,
        grid=(S1 // EBLK,),
        in_specs=[
            pl.BlockSpec((EBLK, DE), lambda i: (i, 0)),
            pl.BlockSpec((DE, D), lambda i: (0, 0)),
            pl.BlockSpec((D, 1), lambda i: (0, 0)),
        ],
        out_specs=[
            pl.BlockSpec((EBLK, D), lambda i: (i, 0)),
            pl.BlockSpec((EBLK, 1), lambda i: (i, 0)),
        ],
        out_shape=[
            jax.ShapeDtypeStruct((S1, D), jnp.float32),
            jax.ShapeDtypeStruct((S1, 1), jnp.float32),
        ],
    )(attrp, wet, aO2)


_MESH = plsc.VectorSubcoreMesh(core_axis_name="c", subcore_axis_name="s")


@functools.partial(
    pl.kernel,
    mesh=_MESH,
    compiler_params=pltpu.CompilerParams(needs_layout_passes=False),
    out_type=[
        jax.ShapeDtypeStruct((S1,), jnp.float32),
        jax.ShapeDtypeStruct((S2,), jnp.float32),
    ],
    scratch_types=[
        pltpu.VMEM((N,), jnp.float32),
        pltpu.VMEM((N,), jnp.float32),
        pltpu.VMEM((N,), jnp.float32),
        pltpu.VMEM((N,), jnp.float32),
        pltpu.VMEM((CH,), jnp.int32),
        pltpu.VMEM((CH,), jnp.int32),
        pltpu.VMEM((CH,), jnp.float32),
        pltpu.VMEM((CH,), jnp.float32),
    ],
)
def _logits_kernel(ares_h, cop_h, p_h, q_h, de_h, src1_h, dst1_h, src2_h, dst2_h,
                   lop_h, lres_h,
                   ares_v, cop_v, p_v, q_v, si_v, di_v, dv_v, lb_v):
    cid = lax.axis_index("c")
    sid = lax.axis_index("s")
    wid = sid * NC + cid
    pltpu.sync_copy(ares_h, ares_v)
    pltpu.sync_copy(cop_h, cop_v)
    pltpu.sync_copy(p_h, p_v)
    pltpu.sync_copy(q_h, q_v)
    lanes = lax.iota(jnp.int32, L)

    def chunk_need(c, _):
        base = wid * E1_WP + c * CH
        pltpu.sync_copy(src1_h.at[pl.ds(base, CH)], si_v)
        pltpu.sync_copy(dst1_h.at[pl.ds(base, CH)], di_v)
        pltpu.sync_copy(de_h.at[pl.ds(base, CH)], dv_v)

        def g16(g, _):
            sl = pl.ds(g * L, L)
            x = (plsc.load_gather(ares_v, [di_v[sl]])
                 + plsc.load_gather(cop_v, [si_v[sl]])
                 + dv_v[sl])
            x = jnp.where(x > 0, x, 0.2 * x)
            pos = c * CH + g * L + lanes
            lb_v[sl] = jnp.where(pos < E1_W, x, -1e30)
            return 0

        lax.fori_loop(0, CH // L, g16, 0)
        pltpu.sync_copy(lb_v, lop_h.at[pl.ds(base, CH)])
        return 0

    lax.fori_loop(0, NCH1, chunk_need, 0)

    def chunk_same(c, _):
        base = wid * E2_WP + c * CH
        pltpu.sync_copy(src2_h.at[pl.ds(base, CH)], si_v)
        pltpu.sync_copy(dst2_h.at[pl.ds(base, CH)], di_v)

        def g16(g, _):
            sl = pl.ds(g * L, L)
            x = (plsc.load_gather(p_v, [di_v[sl]])
                 + plsc.load_gather(q_v, [si_v[sl]]))
            x = jnp.where(x > 0, x, 0.2 * x)
            pos = c * CH + g * L + lanes
            lb_v[sl] = jnp.where(pos < E2_W, x, -1e30)
            return 0

        lax.fori_loop(0, CH // L, g16, 0)
        pltpu.sync_copy(lb_v, lres_h.at[pl.ds(base, CH)])
        return 0

    lax.fori_loop(0, NCH2, chunk_same, 0)


def _max_body(x_ref, m_ref, acc):
    i = pl.program_id(0)

    @pl.when(i == 0)
    def _():
        acc[0, 0] = -jnp.inf

    acc[0, 0] = jnp.maximum(acc[0, 0], jnp.max(x_ref[...]))
    m_ref[0, 0] = acc[0, 0]


def _max_call(lpad):
    return pl.pallas_call(
        _max_body,
        grid=(SM_ROWS // SM_BLK,),
        in_specs=[pl.BlockSpec((SM_BLK, 128), lambda i: (i, 0))],
        out_specs=pl.BlockSpec((1, 1), lambda i: (0, 0), memory_space=pltpu.SMEM),
        out_shape=jax.ShapeDtypeStruct((1, 1), jnp.float32),
        scratch_shapes=[pltpu.SMEM((1, 1), jnp.float32)],
    )(lpad)


def _exp_body(x_ref, m_ref, w_ref, z_ref, acc):
    i = pl.program_id(0)

    @pl.when(i == 0)
    def _():
        acc[0, 0] = 0.0

    w = jnp.exp(x_ref[...] - m_ref[0, 0])
    w_ref[...] = w
    acc[0, 0] = acc[0, 0] + jnp.sum(w)
    z_ref[0, 0] = acc[0, 0]


def _exp_call(lpad, m):
    return pl.pallas_call(
        _exp_body,
        grid=(SM_ROWS // SM_BLK,),
        in_specs=[
            pl.BlockSpec((SM_BLK, 128), lambda i: (i, 0)),
            pl.BlockSpec((1, 1), lambda i: (0, 0), memory_space=pltpu.SMEM),
        ],
        out_specs=[
            pl.BlockSpec((SM_BLK, 128), lambda i: (i, 0)),
            pl.BlockSpec((1, 1), lambda i: (0, 0), memory_space=pltpu.SMEM),
        ],
        out_shape=[
            jax.ShapeDtypeStruct((SM_ROWS, 128), jnp.float32),
            jax.ShapeDtypeStruct((1, 1), jnp.float32),
        ],
        scratch_shapes=[pltpu.SMEM((1, 1), jnp.float32)],
    )(lpad, m)


@functools.partial(
    pl.kernel,
    mesh=_MESH,
    compiler_params=pltpu.CompilerParams(needs_layout_passes=False),
    out_type=[jax.ShapeDtypeStruct((NC, NP, D), jnp.float32)],
    scratch_types=[
        pltpu.VMEM_SHARED((NP, D), jnp.float32),
        pltpu.VMEM((CH, D), jnp.float32),
        pltpu.VMEM((CH, D), jnp.float32),
        pltpu.VMEM((CH,), jnp.int32),
        pltpu.VMEM((CH,), jnp.int32),
        pltpu.VMEM((CH,), jnp.float32),
        pltpu.SemaphoreType.DMA,
    ],
)
def _scatter_kernel(opj_h, rpj_h, aprj_h, src1_h, dst1_h, w1_h, src2_h, dst2_h, w2_h,
                    acc_h,
                    acc_sh, rows_v, apr_v, si_v, di_v, wv_v, sem):
    cid = lax.axis_index("c")
    sid = lax.axis_index("s")
    wid = sid * NC + cid

    def z_rows(t, _):
        rows_v[t // 8, pl.ds((t % 8) * L, L)] = jnp.zeros((L,), jnp.float32)
        return 0

    lax.fori_loop(0, CH * 8, z_rows, 0)

    def z_acc(c, _):
        r0 = sid * ROWS_T + c * CH
        pltpu.sync_copy(rows_v, acc_sh.at[pl.ds(r0, CH)])
        return 0

    lax.fori_loop(0, ROWS_T // CH, z_acc, 0)
    plsc.subcore_barrier()

    def chunk_need(c, _):
        base = wid * E1_WP + c * CH
        pltpu.sync_copy(src1_h.at[pl.ds(base, CH)], si_v)
        pltpu.sync_copy(dst1_h.at[pl.ds(base, CH)], di_v)
        pltpu.sync_copy(w1_h.at[pl.ds(base, CH)], wv_v)
        pltpu.sync_copy(aprj_h.at[pl.ds(base, CH)], apr_v)
        pltpu.async_copy(opj_h.at[si_v], rows_v, sem).wait()

        def scale(e, _):
            we = plsc.load_gather(wv_v, [jnp.full((L,), e, jnp.int32)])
            for j in range(D // L):
                sl = pl.ds(j * L, L)
                rows_v[e, sl] = (rows_v[e, sl] + apr_v[e, sl]) * we
            return 0

        lax.fori_loop(0, CH, scale, 0)
        pltpu.sync_copy(rows_v, acc_sh.at[di_v], add=True)
        return 0

    lax.fori_loop(0, NCH1, chunk_need, 0)

    def chunk_same(c, _):
        base = wid * E2_WP + c * CH
        pltpu.sync_copy(src2_h.at[pl.ds(base, CH)], si_v)
        pltpu.sync_copy(dst2_h.at[pl.ds(base, CH)], di_v)
        pltpu.sync_copy(w2_h.at[pl.ds(base, CH)], wv_v)
        pltpu.async_copy(rpj_h.at[si_v], rows_v, sem).wait()

        def scale(e, _):
            we = plsc.load_gather(wv_v, [jnp.full((L,), e, jnp.int32)])
            for j in range(D // L):
                sl = pl.ds(j * L, L)
                rows_v[e, sl] = rows_v[e, sl] * we
            return 0

        lax.fori_loop(0, CH, scale, 0)
        pltpu.sync_copy(rows_v, acc_sh.at[di_v], add=True)
        return 0

    lax.fori_loop(0, NCH2, chunk_same, 0)
    plsc.subcore_barrier()

    r0 = sid * ROWS_T
    pltpu.sync_copy(acc_sh.at[pl.ds(r0, ROWS_T)], acc_h.at[cid, pl.ds(r0, ROWS_T)])


def _combine_body(r_ref, a0_ref, a1_ref, ws_ref, z_ref, o_ref):
    zinv = 1.0 / z_ref[0, 0]
    x = (r_ref[...] * (ws_ref[...] * zinv)
         + (a0_ref[...] + a1_ref[...]) * zinv)
    o_ref[...] = jnp.where(x > 0, x, jnp.exp(jnp.minimum(x, 0.0)) - 1.0)


def _combine_call(resources, a0, a1, ws, z):
    nblk = N // BLK
    return pl.pallas_call(
        _combine_body,
        grid=(nblk,),
        in_specs=[
            pl.BlockSpec((BLK, D), lambda i: (i, 0)),
            pl.BlockSpec((BLK, D), lambda i: (i, 0)),
            pl.BlockSpec((BLK, D), lambda i: (i, 0)),
            pl.BlockSpec((BLK, 1), lambda i: (i, 0)),
            pl.BlockSpec((1, 1), lambda i: (0, 0), memory_space=pltpu.SMEM),
        ],
        out_specs=pl.BlockSpec((BLK, D), lambda i: (i, 0)),
        out_shape=jax.ShapeDtypeStruct((N, D), jnp.float32),
    )(resources, a0, a1, ws, z)


def _pad_seg(x, ew, ewp):
    if x.ndim == 1:
        return jnp.pad(x.reshape(NW, ew), ((0, 0), (0, ewp - ew))).reshape(NW * ewp)
    return jnp.pad(x.reshape(NW, ew, x.shape[-1]),
                   ((0, 0), (0, ewp - ew), (0, 0))).reshape(NW * ewp, x.shape[-1])


def kernel(resources, operations, need_edge_index, need_edge_attr, same_edge_index,
           W_self, W_res, W_op, att_op, att_res, att_self):
    f32 = jnp.float32
    resources = resources.astype(f32)
    operations = operations.astype(f32)
    need_edge_attr = need_edge_attr.astype(f32)
    src1p = _pad_seg(need_edge_index[0].astype(jnp.int32), E1_W, E1_WP)
    dst1p = _pad_seg(need_edge_index[1].astype(jnp.int32), E1_W, E1_WP)
    src2p = _pad_seg(same_edge_index[0].astype(jnp.int32), E2_W, E2_WP)
    dst2p = _pad_seg(same_edge_index[1].astype(jnp.int32), E2_W, E2_WP)
    attrp = _pad_seg(need_edge_attr, E1_W, E1_WP)

    wst = W_self.T.astype(f32)
    wrt = W_res.T.astype(f32)
    wot = W_op[:, :D].T.astype(f32)
    wet = W_op[:, D:].T.astype(f32)
    aS = (att_self[:D] + att_self[D:]).astype(f32)
    aO1 = att_op[:D].astype(f32)
    aO2 = att_op[D:].astype(f32)
    aR1 = att_res[:D].astype(f32)
    aR2 = att_res[D:].astype(f32)

    res_proj, ops_proj, scal = _dense_call(resources, operations, wst, wrt, wot,
                                           aS, aO1, aO2, aR1, aR2)
    aprj, d_edge = _attr_call(attrp, wet, aO2)
    d_edge = d_edge[:, 0]

    l_self = scal[:, 0]
    a_res = scal[:, 1]
    p = scal[:, 2]
    q = scal[:, 3]
    c_op = scal[:, 4]

    l_op, l_res = _logits_kernel(a_res, c_op, p, q, d_edge,
                                 src1p, dst1p, src2p, dst2p)

    l_all = jnp.concatenate([l_self, l_op, l_res])
    lpad = jnp.pad(l_all, (0, TOTP - TOT), constant_values=-1e30).reshape(SM_ROWS, 128)
    m = _max_call(lpad)
    wpad, z = _exp_call(lpad, m)
    w_all = wpad.reshape(TOTP)
    w_self = w_all[:N]
    w1p = w_all[N:N + S1]
    w2p = w_all[N + S1:N + S1 + S2]

    (acc,) = _scatter_kernel(ops_proj, res_proj, aprj, src1p, dst1p, w1p,
                             src2p, dst2p, w2p)

    return _combine_call(resources, acc[0, :N], acc[1, :N],
                         w_self.reshape(N, 1), z)


# speedup vs baseline: 2.5626x; 1.2137x over previous
import functools

import jax
import jax.numpy as jnp
from jax import lax
from jax.experimental import pallas as pl
from jax.experimental.pallas import tpu as pltpu
from jax.experimental.pallas import tpu_sc as plsc

N = 10000
E1 = 320000
E2 = 160000
D = 128
DE = 16

NC, NS, L = 2, 16, 16
NW = NC * NS

E1_W = E1 // NW
E2_W = E2 // NW
CH = 128
E1_WP = -(-E1_W // (2 * CH)) * (2 * CH)
E2_WP = -(-E2_W // (2 * CH)) * (2 * CH)
NCH1 = E1_WP // CH
NCH2 = E2_WP // CH
S1 = NW * E1_WP
S2 = NW * E2_WP
NP = 10240
ROWS_T = NP // NS

TOT = N + S1 + S2
SM_ROWS = 3920
TOTP = SM_ROWS * 128
SM_BLK = 784

BLK = 400
EBLK = 4096


def _dense_body(r_ref, o_ref, wst_ref, wrt_ref, wot_ref,
                aS_ref, aO1_ref, aO2_ref, aR1_ref, aR2_ref,
                rp_ref, op_ref, scal_ref):
    r = r_ref[...]
    o = o_ref[...]
    sr = jnp.dot(r, wst_ref[...], preferred_element_type=jnp.float32)
    rp = jnp.dot(r, wrt_ref[...], preferred_element_type=jnp.float32)
    op = jnp.dot(o, wot_ref[...], preferred_element_type=jnp.float32)
    rp_ref[...] = rp
    op_ref[...] = op
    s_self = jnp.dot(sr, aS_ref[...])
    l_self = jnp.where(s_self > 0, s_self, 0.2 * s_self)
    a_res = jnp.dot(sr, aO1_ref[...])
    p = jnp.dot(sr, aR1_ref[...])
    q = jnp.dot(rp, aR2_ref[...])
    c_op = jnp.dot(op, aO2_ref[...])
    z = jnp.zeros_like(s_self)
    scal_ref[...] = jnp.concatenate([l_self, a_res, p, q, c_op, z, z, z], axis=1)


def _dense_call(resources, operations, wst, wrt, wot, aS, aO1, aO2, aR1, aR2):
    nblk = N // BLK
    full = lambda s: pl.BlockSpec(s, lambda i: (0, 0))
    return pl.pallas_call(
        _dense_body,
        grid=(nblk,),
        in_specs=[
            pl.BlockSpec((BLK, D), lambda i: (i, 0)),
            pl.BlockSpec((BLK, D), lambda i: (i, 0)),
            full((D, D)), full((D, D)), full((D, D)),
            full((D, 1)), full((D, 1)), full((D, 1)), full((D, 1)), full((D, 1)),
        ],
        out_specs=[
            pl.BlockSpec((BLK, D), lambda i: (i, 0)),
            pl.BlockSpec((BLK, D), lambda i: (i, 0)),
            pl.BlockSpec((BLK, 8), lambda i: (i, 0)),
        ],
        out_shape=[
            jax.ShapeDtypeStruct((N, D), jnp.float32),
            jax.ShapeDtypeStruct((N, D), jnp.float32),
            jax.ShapeDtypeStruct((N, 8), jnp.float32),
        ],
    )(resources, operations, wst, wrt, wot, aS, aO1, aO2, aR1, aR2)


def _attr_body(attr_ref, wet_ref, aO2_ref, ap_ref, d_ref):
    ap = jnp.dot(attr_ref[...], wet_ref[...], preferred_element_type=jnp.float32)
    ap_ref[...] = ap
    d_ref[...] = jnp.dot(ap, aO2_ref[...])


def _attr_call(attrp, wet, aO2):
    return pl.pallas_call(
        _attr_body,
        grid=(S1 // EBLK,),
        in_specs=[
            pl.BlockSpec((EBLK, DE), lambda i: (i, 0)),
            pl.BlockSpec((DE, D), lambda i: (0, 0)),
            pl.BlockSpec((D, 1), lambda i: (0, 0)),
        ],
        out_specs=[
            pl.BlockSpec((EBLK, D), lambda i: (i, 0)),
            pl.BlockSpec((EBLK, 1), lambda i: (i, 0)),
        ],
        out_shape=[
            jax.ShapeDtypeStruct((S1, D), jnp.float32),
            jax.ShapeDtypeStruct((S1, 1), jnp.float32),
        ],
    )(attrp, wet, aO2)


_MESH = plsc.VectorSubcoreMesh(core_axis_name="c", subcore_axis_name="s")


@functools.partial(
    pl.kernel,
    mesh=_MESH,
    compiler_params=pltpu.CompilerParams(needs_layout_passes=False),
    out_type=[
        jax.ShapeDtypeStruct((S1,), jnp.float32),
        jax.ShapeDtypeStruct((S2,), jnp.float32),
    ],
    scratch_types=[
        pltpu.VMEM((N,), jnp.float32),
        pltpu.VMEM((N,), jnp.float32),
        pltpu.VMEM((N,), jnp.float32),
        pltpu.VMEM((N,), jnp.float32),
        pltpu.VMEM((CH,), jnp.int32),
        pltpu.VMEM((CH,), jnp.int32),
        pltpu.VMEM((CH,), jnp.float32),
        pltpu.VMEM((CH,), jnp.float32),
    ],
)
def _logits_kernel(ares_h, cop_h, p_h, q_h, de_h, src1_h, dst1_h, src2_h, dst2_h,
                   lop_h, lres_h,
                   ares_v, cop_v, p_v, q_v, si_v, di_v, dv_v, lb_v):
    cid = lax.axis_index("c")
    sid = lax.axis_index("s")
    wid = sid * NC + cid
    pltpu.sync_copy(ares_h, ares_v)
    pltpu.sync_copy(cop_h, cop_v)
    pltpu.sync_copy(p_h, p_v)
    pltpu.sync_copy(q_h, q_v)
    lanes = lax.iota(jnp.int32, L)

    def chunk_need(c, _):
        base = wid * E1_WP + c * CH
        pltpu.sync_copy(src1_h.at[pl.ds(base, CH)], si_v)
        pltpu.sync_copy(dst1_h.at[pl.ds(base, CH)], di_v)
        pltpu.sync_copy(de_h.at[pl.ds(base, CH)], dv_v)

        def g16(g, _):
            sl = pl.ds(g * L, L)
            x = (plsc.load_gather(ares_v, [di_v[sl]])
                 + plsc.load_gather(cop_v, [si_v[sl]])
                 + dv_v[sl])
            x = jnp.where(x > 0, x, 0.2 * x)
            pos = c * CH + g * L + lanes
            lb_v[sl] = jnp.where(pos < E1_W, x, -1e30)
            return 0

        lax.fori_loop(0, CH // L, g16, 0)
        pltpu.sync_copy(lb_v, lop_h.at[pl.ds(base, CH)])
        return 0

    lax.fori_loop(0, NCH1, chunk_need, 0)

    def chunk_same(c, _):
        base = wid * E2_WP + c * CH
        pltpu.sync_copy(src2_h.at[pl.ds(base, CH)], si_v)
        pltpu.sync_copy(dst2_h.at[pl.ds(base, CH)], di_v)

        def g16(g, _):
            sl = pl.ds(g * L, L)
            x = (plsc.load_gather(p_v, [di_v[sl]])
                 + plsc.load_gather(q_v, [si_v[sl]]))
            x = jnp.where(x > 0, x, 0.2 * x)
            pos = c * CH + g * L + lanes
            lb_v[sl] = jnp.where(pos < E2_W, x, -1e30)
            return 0

        lax.fori_loop(0, CH // L, g16, 0)
        pltpu.sync_copy(lb_v, lres_h.at[pl.ds(base, CH)])
        return 0

    lax.fori_loop(0, NCH2, chunk_same, 0)


def _max_body(x_ref, m_ref, acc):
    i = pl.program_id(0)

    @pl.when(i == 0)
    def _():
        acc[0, 0] = -jnp.inf

    acc[0, 0] = jnp.maximum(acc[0, 0], jnp.max(x_ref[...]))
    m_ref[0, 0] = acc[0, 0]


def _max_call(lpad):
    return pl.pallas_call(
        _max_body,
        grid=(SM_ROWS // SM_BLK,),
        in_specs=[pl.BlockSpec((SM_BLK, 128), lambda i: (i, 0))],
        out_specs=pl.BlockSpec((1, 1), lambda i: (0, 0), memory_space=pltpu.SMEM),
        out_shape=jax.ShapeDtypeStruct((1, 1), jnp.float32),
        scratch_shapes=[pltpu.SMEM((1, 1), jnp.float32)],
    )(lpad)


def _exp_body(x_ref, m_ref, w_ref, z_ref, acc):
    i = pl.program_id(0)

    @pl.when(i == 0)
    def _():
        acc[0, 0] = 0.0

    w = jnp.exp(x_ref[...] - m_ref[0, 0])
    w_ref[...] = w
    acc[0, 0] = acc[0, 0] + jnp.sum(w)
    z_ref[0, 0] = acc[0, 0]


def _exp_call(lpad, m):
    return pl.pallas_call(
        _exp_body,
        grid=(SM_ROWS // SM_BLK,),
        in_specs=[
            pl.BlockSpec((SM_BLK, 128), lambda i: (i, 0)),
            pl.BlockSpec((1, 1), lambda i: (0, 0), memory_space=pltpu.SMEM),
        ],
        out_specs=[
            pl.BlockSpec((SM_BLK, 128), lambda i: (i, 0)),
            pl.BlockSpec((1, 1), lambda i: (0, 0), memory_space=pltpu.SMEM),
        ],
        out_shape=[
            jax.ShapeDtypeStruct((SM_ROWS, 128), jnp.float32),
            jax.ShapeDtypeStruct((1, 1), jnp.float32),
        ],
        scratch_shapes=[pltpu.SMEM((1, 1), jnp.float32)],
    )(lpad, m)


@functools.partial(
    pl.kernel,
    mesh=_MESH,
    compiler_params=pltpu.CompilerParams(needs_layout_passes=False),
    out_type=[jax.ShapeDtypeStruct((NC, NP, D), jnp.float32)],
    scratch_types=[
        pltpu.VMEM_SHARED((NP, D), jnp.float32),
        pltpu.VMEM((CH, D), jnp.float32),
        pltpu.VMEM((CH, D), jnp.float32),
        pltpu.VMEM((CH,), jnp.int32),
        pltpu.VMEM((CH,), jnp.int32),
        pltpu.VMEM((CH,), jnp.int32),
        pltpu.VMEM((CH,), jnp.int32),
        pltpu.VMEM((CH,), jnp.float32),
        pltpu.VMEM((CH,), jnp.float32),
        pltpu.SemaphoreType.DMA,
        pltpu.SemaphoreType.DMA,
    ],
)
def _scatter_kernel(opj_h, rpj_h, aprj_h, src1_h, dst1_h, w1_h, src2_h, dst2_h, w2_h,
                    acc_h,
                    acc_sh, buf0, buf1, si0, si1, di0, di1, wv0, wv1, sem0, sem1):
    cid = lax.axis_index("c")
    sid = lax.axis_index("s")
    wid = sid * NC + cid
    bufs, sis, dis, wvs, sems = (buf0, buf1), (si0, si1), (di0, di1), (wv0, wv1), (sem0, sem1)

    def z_rows(t, _):
        buf0[t // 8, pl.ds((t % 8) * L, L)] = jnp.zeros((L,), jnp.float32)
        return 0

    lax.fori_loop(0, CH * 8, z_rows, 0)

    def z_acc(c, _):
        r0 = sid * ROWS_T + c * CH
        pltpu.sync_copy(buf0, acc_sh.at[pl.ds(r0, CH)])
        return 0

    lax.fori_loop(0, ROWS_T // CH, z_acc, 0)
    plsc.subcore_barrier()

    def run_edges(tbl_h, src_h, dst_h, w_h, apr_h, nch, ewp):
        def issue(c, b):
            base = wid * ewp + c * CH
            pltpu.sync_copy(src_h.at[pl.ds(base, CH)], sis[b])
            pltpu.sync_copy(dst_h.at[pl.ds(base, CH)], dis[b])
            pltpu.sync_copy(w_h.at[pl.ds(base, CH)], wvs[b])
            if apr_h is not None:
                pltpu.sync_copy(apr_h.at[pl.ds(base, CH)], bufs[b])
                pltpu.async_copy(tbl_h.at[sis[b]], bufs[b], sems[b], add=True)
            else:
                pltpu.async_copy(tbl_h.at[sis[b]], bufs[b], sems[b])

        issue(0, 0)
        issue(1, 1)

        def outer(g, _):
            for b in range(2):
                c = 2 * g + b
                pltpu.make_async_copy(tbl_h.at[sis[b]], bufs[b], sems[b]).wait()

                def scale(e, _, b=b):
                    we = plsc.load_gather(wvs[b], [jnp.full((L,), e, jnp.int32)])
                    for j in range(D // L):
                        sl = pl.ds(j * L, L)
                        bufs[b][e, sl] = bufs[b][e, sl] * we
                    return 0

                lax.fori_loop(0, CH, scale, 0)
                pltpu.sync_copy(bufs[b], acc_sh.at[dis[b]], add=True)

                @pl.when(c + 2 < nch)
                def _(c=c, b=b):
                    issue(c + 2, b)
            return 0

        lax.fori_loop(0, nch // 2, outer, 0)

    run_edges(opj_h, src1_h, dst1_h, w1_h, aprj_h, NCH1, E1_WP)
    run_edges(rpj_h, src2_h, dst2_h, w2_h, None, NCH2, E2_WP)
    plsc.subcore_barrier()

    r0 = sid * ROWS_T
    pltpu.sync_copy(acc_sh.at[pl.ds(r0, ROWS_T)], acc_h.at[cid, pl.ds(r0, ROWS_T)])


def _combine_body(r_ref, a0_ref, a1_ref, ws_ref, z_ref, o_ref):
    zinv = 1.0 / z_ref[0, 0]
    x = (r_ref[...] * (ws_ref[...] * zinv)
         + (a0_ref[...] + a1_ref[...]) * zinv)
    o_ref[...] = jnp.where(x > 0, x, jnp.exp(jnp.minimum(x, 0.0)) - 1.0)


def _combine_call(resources, a0, a1, ws, z):
    nblk = N // BLK
    return pl.pallas_call(
        _combine_body,
        grid=(nblk,),
        in_specs=[
            pl.BlockSpec((BLK, D), lambda i: (i, 0)),
            pl.BlockSpec((BLK, D), lambda i: (i, 0)),
            pl.BlockSpec((BLK, D), lambda i: (i, 0)),
            pl.BlockSpec((BLK, 1), lambda i: (i, 0)),
            pl.BlockSpec((1, 1), lambda i: (0, 0), memory_space=pltpu.SMEM),
        ],
        out_specs=pl.BlockSpec((BLK, D), lambda i: (i, 0)),
        out_shape=jax.ShapeDtypeStruct((N, D), jnp.float32),
    )(resources, a0, a1, ws, z)


def _pad_seg(x, ew, ewp):
    if x.ndim == 1:
        return jnp.pad(x.reshape(NW, ew), ((0, 0), (0, ewp - ew))).reshape(NW * ewp)
    return jnp.pad(x.reshape(NW, ew, x.shape[-1]),
                   ((0, 0), (0, ewp - ew), (0, 0))).reshape(NW * ewp, x.shape[-1])


def kernel(resources, operations, need_edge_index, need_edge_attr, same_edge_index,
           W_self, W_res, W_op, att_op, att_res, att_self):
    f32 = jnp.float32
    resources = resources.astype(f32)
    operations = operations.astype(f32)
    need_edge_attr = need_edge_attr.astype(f32)
    src1p = _pad_seg(need_edge_index[0].astype(jnp.int32), E1_W, E1_WP)
    dst1p = _pad_seg(need_edge_index[1].astype(jnp.int32), E1_W, E1_WP)
    src2p = _pad_seg(same_edge_index[0].astype(jnp.int32), E2_W, E2_WP)
    dst2p = _pad_seg(same_edge_index[1].astype(jnp.int32), E2_W, E2_WP)
    attrp = _pad_seg(need_edge_attr, E1_W, E1_WP)

    wst = W_self.T.astype(f32)
    wrt = W_res.T.astype(f32)
    wot = W_op[:, :D].T.astype(f32)
    wet = W_op[:, D:].T.astype(f32)
    aS = (att_self[:D] + att_self[D:]).astype(f32)
    aO1 = att_op[:D].astype(f32)
    aO2 = att_op[D:].astype(f32)
    aR1 = att_res[:D].astype(f32)
    aR2 = att_res[D:].astype(f32)

    res_proj, ops_proj, scal = _dense_call(resources, operations, wst, wrt, wot,
                                           aS, aO1, aO2, aR1, aR2)
    aprj, d_edge = _attr_call(attrp, wet, aO2)
    d_edge = d_edge[:, 0]

    l_self = scal[:, 0]
    a_res = scal[:, 1]
    p = scal[:, 2]
    q = scal[:, 3]
    c_op = scal[:, 4]

    l_op, l_res = _logits_kernel(a_res, c_op, p, q, d_edge,
                                 src1p, dst1p, src2p, dst2p)

    l_all = jnp.concatenate([l_self, l_op, l_res])
    lpad = jnp.pad(l_all, (0, TOTP - TOT), constant_values=-1e30).reshape(SM_ROWS, 128)
    m = _max_call(lpad)
    wpad, z = _exp_call(lpad, m)
    w_all = wpad.reshape(TOTP)
    w_self = w_all[:N]
    w1p = w_all[N:N + S1]
    w2p = w_all[N + S1:N + S1 + S2]

    (acc,) = _scatter_kernel(ops_proj, res_proj, aprj, src1p, dst1p, w1p,
                             src2p, dst2p, w2p)

    return _combine_call(resources, acc[0, :N], acc[1, :N],
                         w_self.reshape(N, 1), z)


# speedup vs baseline: 2.7661x; 1.0794x over previous
import functools

import jax
import jax.numpy as jnp
from jax import lax
from jax.experimental import pallas as pl
from jax.experimental.pallas import tpu as pltpu
from jax.experimental.pallas import tpu_sc as plsc

N = 10000
E1 = 320000
E2 = 160000
D = 128
DE = 16

NC, NS, L = 2, 16, 16
NW = NC * NS

E1_W = E1 // NW
E2_W = E2 // NW
CH = 128
E1_WP = -(-E1_W // (2 * CH)) * (2 * CH)
E2_WP = -(-E2_W // (2 * CH)) * (2 * CH)
NCH1 = E1_WP // CH
NCH2 = E2_WP // CH
S1 = NW * E1_WP
S2 = NW * E2_WP
NP = 10240
ROWS_T = NP // NS

TOT = N + S1 + S2
SM_ROWS = 3920
TOTP = SM_ROWS * 128
SM_BLK = 784

BLK = 400
EBLK = 4096


def _dense_body(r_ref, o_ref, wst_ref, wrt_ref, wot_ref,
                aS_ref, aO1_ref, aO2_ref, aR1_ref, aR2_ref,
                rp_ref, op_ref, scal_ref):
    r = r_ref[...]
    o = o_ref[...]
    sr = jnp.dot(r, wst_ref[...], preferred_element_type=jnp.float32)
    rp = jnp.dot(r, wrt_ref[...], preferred_element_type=jnp.float32)
    op = jnp.dot(o, wot_ref[...], preferred_element_type=jnp.float32)
    rp_ref[...] = rp
    op_ref[...] = op
    s_self = jnp.dot(sr, aS_ref[...])
    l_self = jnp.where(s_self > 0, s_self, 0.2 * s_self)
    a_res = jnp.dot(sr, aO1_ref[...])
    p = jnp.dot(sr, aR1_ref[...])
    q = jnp.dot(rp, aR2_ref[...])
    c_op = jnp.dot(op, aO2_ref[...])
    z = jnp.zeros_like(s_self)
    scal_ref[...] = jnp.concatenate([l_self, a_res, p, q, c_op, z, z, z], axis=1)


def _dense_call(resources, operations, wst, wrt, wot, aS, aO1, aO2, aR1, aR2):
    nblk = N // BLK
    full = lambda s: pl.BlockSpec(s, lambda i: (0, 0))
    return pl.pallas_call(
        _dense_body,
        grid=(nblk,),
        in_specs=[
            pl.BlockSpec((BLK, D), lambda i: (i, 0)),
            pl.BlockSpec((BLK, D), lambda i: (i, 0)),
            full((D, D)), full((D, D)), full((D, D)),
            full((D, 1)), full((D, 1)), full((D, 1)), full((D, 1)), full((D, 1)),
        ],
        out_specs=[
            pl.BlockSpec((BLK, D), lambda i: (i, 0)),
            pl.BlockSpec((BLK, D), lambda i: (i, 0)),
            pl.BlockSpec((BLK, 8), lambda i: (i, 0)),
        ],
        out_shape=[
            jax.ShapeDtypeStruct((N, D), jnp.float32),
            jax.ShapeDtypeStruct((N, D), jnp.float32),
            jax.ShapeDtypeStruct((N, 8), jnp.float32),
        ],
    )(resources, operations, wst, wrt, wot, aS, aO1, aO2, aR1, aR2)


def _attr_body(attr_ref, wet_ref, aO2_ref, ap_ref, d_ref):
    ap = jnp.dot(attr_ref[...], wet_ref[...], preferred_element_type=jnp.float32)
    ap_ref[...] = ap
    d_ref[...] = jnp.dot(ap, aO2_ref[...])


def _attr_call(attrp, wet, aO2):
    return pl.pallas_call(
        _attr_body,
        grid=(S1 // EBLK,),
        in_specs=[
            pl.BlockSpec((EBLK, DE), lambda i: (i, 0)),
            pl.BlockSpec((DE, D), lambda i: (0, 0)),
            pl.BlockSpec((D, 1), lambda i: (0, 0)),
        ],
        out_specs=[
            pl.BlockSpec((EBLK, D), lambda i: (i, 0)),
            pl.BlockSpec((EBLK, 1), lambda i: (i, 0)),
        ],
        out_shape=[
            jax.ShapeDtypeStruct((S1, D), jnp.float32),
            jax.ShapeDtypeStruct((S1, 1), jnp.float32),
        ],
    )(attrp, wet, aO2)


_MESH = plsc.VectorSubcoreMesh(core_axis_name="c", subcore_axis_name="s")


@functools.partial(
    pl.kernel,
    mesh=_MESH,
    compiler_params=pltpu.CompilerParams(needs_layout_passes=False),
    out_type=[
        jax.ShapeDtypeStruct((S1,), jnp.float32),
        jax.ShapeDtypeStruct((S2,), jnp.float32),
    ],
    scratch_types=[
        pltpu.VMEM((N,), jnp.float32),
        pltpu.VMEM((N,), jnp.float32),
        pltpu.VMEM((N,), jnp.float32),
        pltpu.VMEM((N,), jnp.float32),
        pltpu.VMEM((CH,), jnp.int32),
        pltpu.VMEM((CH,), jnp.int32),
        pltpu.VMEM((CH,), jnp.float32),
        pltpu.VMEM((CH,), jnp.float32),
    ],
)
def _logits_kernel(ares_h, cop_h, p_h, q_h, de_h, src1_h, dst1_h, src2_h, dst2_h,
                   lop_h, lres_h,
                   ares_v, cop_v, p_v, q_v, si_v, di_v, dv_v, lb_v):
    cid = lax.axis_index("c")
    sid = lax.axis_index("s")
    wid = sid * NC + cid
    pltpu.sync_copy(ares_h, ares_v)
    pltpu.sync_copy(cop_h, cop_v)
    pltpu.sync_copy(p_h, p_v)
    pltpu.sync_copy(q_h, q_v)
    lanes = lax.iota(jnp.int32, L)

    def chunk_need(c, _):
        base = wid * E1_WP + c * CH
        pltpu.sync_copy(src1_h.at[pl.ds(base, CH)], si_v)
        pltpu.sync_copy(dst1_h.at[pl.ds(base, CH)], di_v)
        pltpu.sync_copy(de_h.at[pl.ds(base, CH)], dv_v)

        def g16(g, _):
            sl = pl.ds(g * L, L)
            x = (plsc.load_gather(ares_v, [di_v[sl]])
                 + plsc.load_gather(cop_v, [si_v[sl]])
                 + dv_v[sl])
            x = jnp.where(x > 0, x, 0.2 * x)
            pos = c * CH + g * L + lanes
            lb_v[sl] = jnp.where(pos < E1_W, x, -1e30)
            return 0

        lax.fori_loop(0, CH // L, g16, 0)
        pltpu.sync_copy(lb_v, lop_h.at[pl.ds(base, CH)])
        return 0

    lax.fori_loop(0, NCH1, chunk_need, 0)

    def chunk_same(c, _):
        base = wid * E2_WP + c * CH
        pltpu.sync_copy(src2_h.at[pl.ds(base, CH)], si_v)
        pltpu.sync_copy(dst2_h.at[pl.ds(base, CH)], di_v)

        def g16(g, _):
            sl = pl.ds(g * L, L)
            x = (plsc.load_gather(p_v, [di_v[sl]])
                 + plsc.load_gather(q_v, [si_v[sl]]))
            x = jnp.where(x > 0, x, 0.2 * x)
            pos = c * CH + g * L + lanes
            lb_v[sl] = jnp.where(pos < E2_W, x, -1e30)
            return 0

        lax.fori_loop(0, CH // L, g16, 0)
        pltpu.sync_copy(lb_v, lres_h.at[pl.ds(base, CH)])
        return 0

    lax.fori_loop(0, NCH2, chunk_same, 0)


def _max_body(x_ref, m_ref, acc):
    i = pl.program_id(0)

    @pl.when(i == 0)
    def _():
        acc[0, 0] = -jnp.inf

    acc[0, 0] = jnp.maximum(acc[0, 0], jnp.max(x_ref[...]))
    m_ref[0, 0] = acc[0, 0]


def _max_call(lpad):
    return pl.pallas_call(
        _max_body,
        grid=(SM_ROWS // SM_BLK,),
        in_specs=[pl.BlockSpec((SM_BLK, 128), lambda i: (i, 0))],
        out_specs=pl.BlockSpec((1, 1), lambda i: (0, 0), memory_space=pltpu.SMEM),
        out_shape=jax.ShapeDtypeStruct((1, 1), jnp.float32),
        scratch_shapes=[pltpu.SMEM((1, 1), jnp.float32)],
    )(lpad)


def _exp_body(x_ref, m_ref, w_ref, z_ref, acc):
    i = pl.program_id(0)

    @pl.when(i == 0)
    def _():
        acc[0, 0] = 0.0

    w = jnp.exp(x_ref[...] - m_ref[0, 0])
    w_ref[...] = w
    acc[0, 0] = acc[0, 0] + jnp.sum(w)
    z_ref[0, 0] = acc[0, 0]


def _exp_call(lpad, m):
    return pl.pallas_call(
        _exp_body,
        grid=(SM_ROWS // SM_BLK,),
        in_specs=[
            pl.BlockSpec((SM_BLK, 128), lambda i: (i, 0)),
            pl.BlockSpec((1, 1), lambda i: (0, 0), memory_space=pltpu.SMEM),
        ],
        out_specs=[
            pl.BlockSpec((SM_BLK, 128), lambda i: (i, 0)),
            pl.BlockSpec((1, 1), lambda i: (0, 0), memory_space=pltpu.SMEM),
        ],
        out_shape=[
            jax.ShapeDtypeStruct((SM_ROWS, 128), jnp.float32),
            jax.ShapeDtypeStruct((1, 1), jnp.float32),
        ],
        scratch_shapes=[pltpu.SMEM((1, 1), jnp.float32)],
    )(lpad, m)


@functools.partial(
    pl.kernel,
    mesh=_MESH,
    compiler_params=pltpu.CompilerParams(needs_layout_passes=False),
    out_type=[jax.ShapeDtypeStruct((NC, NP, D), jnp.float32)],
    scratch_types=[
        pltpu.VMEM_SHARED((NP, D), jnp.float32),
        pltpu.VMEM((CH, D), jnp.float32),
        pltpu.VMEM((CH, D), jnp.float32),
        pltpu.VMEM((NCH1, CH), jnp.int32),
        pltpu.VMEM((NCH2, CH), jnp.int32),
        pltpu.VMEM((CH,), jnp.int32),
        pltpu.VMEM((CH,), jnp.int32),
        pltpu.VMEM((CH,), jnp.float32),
        pltpu.VMEM((CH,), jnp.float32),
        pltpu.SemaphoreType.DMA,
        pltpu.SemaphoreType.DMA,
        pltpu.SemaphoreType.DMA,
        pltpu.SemaphoreType.DMA,
    ],
)
def _scatter_kernel(opj_h, rpj_h, aprj_h, src1_h, dst1_h, w1_h, src2_h, dst2_h, w2_h,
                    acc_h,
                    acc_sh, buf0, buf1, di1_a, di2_a, si0, si1, wv0, wv1,
                    sem0, sem1, ssem0, ssem1):
    cid = lax.axis_index("c")
    sid = lax.axis_index("s")
    wid = sid * NC + cid
    bufs, sis, wvs = (buf0, buf1), (si0, si1), (wv0, wv1)
    sems, ssems = (sem0, sem1), (ssem0, ssem1)

    pltpu.sync_copy(dst1_h.at[pl.ds(wid * NCH1, NCH1)], di1_a)
    pltpu.sync_copy(dst2_h.at[pl.ds(wid * NCH2, NCH2)], di2_a)

    def z_rows(t, _):
        buf0[t // 8, pl.ds((t % 8) * L, L)] = jnp.zeros((L,), jnp.float32)
        return 0

    lax.fori_loop(0, CH * 8, z_rows, 0)

    def z_acc(c, _):
        r0 = sid * ROWS_T + c * CH
        pltpu.sync_copy(buf0, acc_sh.at[pl.ds(r0, CH)])
        return 0

    lax.fori_loop(0, ROWS_T // CH, z_acc, 0)
    plsc.subcore_barrier()

    def run_edges(tbl_h, src_h, w_h, di_a, apr_h, nch, nch_rows_base):
        def issue(c, b, first):
            row = wid * nch_rows_base + c
            pltpu.sync_copy(src_h.at[row], sis[b])
            pltpu.sync_copy(w_h.at[row], wvs[b])
            if not first:
                pltpu.make_async_copy(bufs[b], acc_sh.at[di_a.at[c]], ssems[b]).wait()
            if apr_h is not None:
                pltpu.sync_copy(apr_h.at[pl.ds(row * CH, CH)], bufs[b])
                pltpu.async_copy(tbl_h.at[sis[b]], bufs[b], sems[b], add=True)
            else:
                pltpu.async_copy(tbl_h.at[sis[b]], bufs[b], sems[b])

        issue(0, 0, True)
        issue(1, 1, True)

        def outer(g, _):
            for b in range(2):
                c = 2 * g + b
                pltpu.make_async_copy(tbl_h.at[sis[b]], bufs[b], sems[b]).wait()

                def scale(i, _, b=b):
                    for u in range(2):
                        e = 2 * i + u
                        we = plsc.load_gather(wvs[b], [jnp.full((L,), e, jnp.int32)])
                        for j in range(D // L):
                            sl = pl.ds(j * L, L)
                            bufs[b][e, sl] = bufs[b][e, sl] * we
                    return 0

                lax.fori_loop(0, CH // 2, scale, 0)
                last = c + 2 >= nch

                @pl.when(jnp.logical_not(last))
                def _(c=c, b=b):
                    pltpu.async_copy(bufs[b], acc_sh.at[di_a.at[c]], ssems[b], add=True)
                    issue(c + 2, b, False)

                @pl.when(last)
                def _(c=c, b=b):
                    pltpu.sync_copy(bufs[b], acc_sh.at[di_a.at[c]], add=True)
            return 0

        lax.fori_loop(0, nch // 2, outer, 0)

    run_edges(opj_h, src1_h, w1_h, di1_a, aprj_h, NCH1, NCH1)
    run_edges(rpj_h, src2_h, w2_h, di2_a, None, NCH2, NCH2)
    plsc.subcore_barrier()

    r0 = sid * ROWS_T
    pltpu.sync_copy(acc_sh.at[pl.ds(r0, ROWS_T)], acc_h.at[cid, pl.ds(r0, ROWS_T)])


def _combine_body(r_ref, a0_ref, a1_ref, ws_ref, z_ref, o_ref):
    zinv = 1.0 / z_ref[0, 0]
    x = (r_ref[...] * (ws_ref[...] * zinv)
         + (a0_ref[...] + a1_ref[...]) * zinv)
    o_ref[...] = jnp.where(x > 0, x, jnp.exp(jnp.minimum(x, 0.0)) - 1.0)


def _combine_call(resources, a0, a1, ws, z):
    nblk = N // BLK
    return pl.pallas_call(
        _combine_body,
        grid=(nblk,),
        in_specs=[
            pl.BlockSpec((BLK, D), lambda i: (i, 0)),
            pl.BlockSpec((BLK, D), lambda i: (i, 0)),
            pl.BlockSpec((BLK, D), lambda i: (i, 0)),
            pl.BlockSpec((BLK, 1), lambda i: (i, 0)),
            pl.BlockSpec((1, 1), lambda i: (0, 0), memory_space=pltpu.SMEM),
        ],
        out_specs=pl.BlockSpec((BLK, D), lambda i: (i, 0)),
        out_shape=jax.ShapeDtypeStruct((N, D), jnp.float32),
    )(resources, a0, a1, ws, z)


def _pad_seg(x, ew, ewp):
    if x.ndim == 1:
        return jnp.pad(x.reshape(NW, ew), ((0, 0), (0, ewp - ew))).reshape(NW * ewp)
    return jnp.pad(x.reshape(NW, ew, x.shape[-1]),
                   ((0, 0), (0, ewp - ew), (0, 0))).reshape(NW * ewp, x.shape[-1])


def kernel(resources, operations, need_edge_index, need_edge_attr, same_edge_index,
           W_self, W_res, W_op, att_op, att_res, att_self):
    f32 = jnp.float32
    resources = resources.astype(f32)
    operations = operations.astype(f32)
    need_edge_attr = need_edge_attr.astype(f32)
    src1p = _pad_seg(need_edge_index[0].astype(jnp.int32), E1_W, E1_WP)
    dst1p = _pad_seg(need_edge_index[1].astype(jnp.int32), E1_W, E1_WP)
    src2p = _pad_seg(same_edge_index[0].astype(jnp.int32), E2_W, E2_WP)
    dst2p = _pad_seg(same_edge_index[1].astype(jnp.int32), E2_W, E2_WP)
    attrp = _pad_seg(need_edge_attr, E1_W, E1_WP)

    wst = W_self.T.astype(f32)
    wrt = W_res.T.astype(f32)
    wot = W_op[:, :D].T.astype(f32)
    wet = W_op[:, D:].T.astype(f32)
    aS = (att_self[:D] + att_self[D:]).astype(f32)
    aO1 = att_op[:D].astype(f32)
    aO2 = att_op[D:].astype(f32)
    aR1 = att_res[:D].astype(f32)
    aR2 = att_res[D:].astype(f32)

    res_proj, ops_proj, scal = _dense_call(resources, operations, wst, wrt, wot,
                                           aS, aO1, aO2, aR1, aR2)
    aprj, d_edge = _attr_call(attrp, wet, aO2)
    d_edge = d_edge[:, 0]

    l_self = scal[:, 0]
    a_res = scal[:, 1]
    p = scal[:, 2]
    q = scal[:, 3]
    c_op = scal[:, 4]

    l_op, l_res = _logits_kernel(a_res, c_op, p, q, d_edge,
                                 src1p, dst1p, src2p, dst2p)

    l_all = jnp.concatenate([l_self, l_op, l_res])
    lpad = jnp.pad(l_all, (0, TOTP - TOT), constant_values=-1e30).reshape(SM_ROWS, 128)
    m = _max_call(lpad)
    wpad, z = _exp_call(lpad, m)
    w_all = wpad.reshape(TOTP)
    w_self = w_all[:N]
    w1p = w_all[N:N + S1]
    w2p = w_all[N + S1:N + S1 + S2]

    (acc,) = _scatter_kernel(ops_proj, res_proj, aprj,
                             src1p.reshape(NW * NCH1, CH),
                             dst1p.reshape(NW * NCH1, CH),
                             w1p.reshape(NW * NCH1, CH),
                             src2p.reshape(NW * NCH2, CH),
                             dst2p.reshape(NW * NCH2, CH),
                             w2p.reshape(NW * NCH2, CH))

    return _combine_call(resources, acc[0, :N], acc[1, :N],
                         w_self.reshape(N, 1), z)
